# baseline (speedup 1.0000x reference)
import functools

import jax
import jax.numpy as jnp
from jax import lax
from jax.experimental import pallas as pl
from jax.experimental.pallas import tpu as pltpu

_MIN_CONF = 0.25
_IOU_THR = 0.45
_N = 20000
_K = 1000
_KP = 1024
_GAIN = min(640 / 1080, 640 / 1920)
_PAD_X = (640 - 1920 * _GAIN) / 2.0
_PAD_Y = (640 - 1080 * _GAIN) / 2.0

_ABLK = 2000
_ANB = _N // _ABLK


def _phase_a_body(pred_ref, out_ref):
    x = pred_ref[...]
    cls = x[:, 5:85]
    m = jnp.max(cls, axis=1, keepdims=True)
    il = lax.broadcasted_iota(jnp.int32, (_ABLK, 80), 1)
    cid = jnp.min(jnp.where(cls == m, il, jnp.int32(2**30)), axis=1,
                  keepdims=True)
    obj = x[:, 4:5]
    conf = obj * m
    cvalid = (cid <= 3) | (cid == 5) | (cid == 7)
    keep = (conf >= _MIN_CONF) & cvalid
    score = jnp.where(keep, conf, -1.0)
    xc = x[:, 0:1]
    yc = x[:, 1:2]
    w = x[:, 2:3]
    h = x[:, 3:4]
    l = jnp.round((xc - w / 2.0 - _PAD_X) / _GAIN)
    t = jnp.round((yc - h / 2.0 - _PAD_Y) / _GAIN)
    r = jnp.round((xc + w / 2.0 - _PAD_X) / _GAIN)
    b = jnp.round((yc + h / 2.0 - _PAD_Y) / _GAIN)
    out_ref[...] = jnp.concatenate(
        [score, cid.astype(jnp.float32), l, t, r, b, score, score], axis=1)


def _phase_a(pred):
    return pl.pallas_call(
        _phase_a_body,
        grid=(_ANB,),
        in_specs=[pl.BlockSpec((_ABLK, 85), lambda i: (i, 0))],
        out_specs=pl.BlockSpec((_ABLK, 8), lambda i: (i, 0)),
        out_shape=jax.ShapeDtypeStruct((_N, 8), jnp.float32),
    )(pred)


_NBLK = 8
_BROWS = _KP // _NBLK


def _nms_body(tb_col_ref, tc_col_ref, tbT_ref, tcf_ref, topv_ref,
              keep_ref, fs_ref, B_ref):
    pid = pl.program_id(0)

    @pl.when(pid < _NBLK)
    def _build():
        r0 = pid * _BROWS
        tc_c = tc_col_ref[pl.ds(r0, _BROWS), :] * 10000.0
        Lc = tb_col_ref[pl.ds(r0, _BROWS), 0:1] + tc_c
        Tc = tb_col_ref[pl.ds(r0, _BROWS), 1:2] + tc_c
        Rc = tb_col_ref[pl.ds(r0, _BROWS), 2:3] + tc_c
        Bc = tb_col_ref[pl.ds(r0, _BROWS), 3:4] + tc_c
        area_c = jnp.maximum(Rc - Lc, 0.0) * jnp.maximum(Bc - Tc, 0.0)
        tc_r = tcf_ref[...] * 10000.0
        Lr = tbT_ref[0:1, :] + tc_r
        Tr = tbT_ref[1:2, :] + tc_r
        Rr = tbT_ref[2:3, :] + tc_r
        Br = tbT_ref[3:4, :] + tc_r
        area_r = jnp.maximum(Rr - Lr, 0.0) * jnp.maximum(Br - Tr, 0.0)
        wx = jnp.clip(jnp.minimum(Rc, Rr) - jnp.maximum(Lc, Lr), 0.0, None)
        wy = jnp.clip(jnp.minimum(Bc, Br) - jnp.maximum(Tc, Tr), 0.0, None)
        inter = wx * wy
        iou = inter / (area_c + area_r - inter + 1e-9)
        ii = lax.broadcasted_iota(jnp.int32, (_BROWS, _KP), 0) + r0
        jj = lax.broadcasted_iota(jnp.int32, (_BROWS, _KP), 1)
        B_ref[pl.ds(r0, _BROWS), :] = (
            (iou > _IOU_THR) & (jj > ii)).astype(jnp.float32)

    @pl.when(pid == _NBLK)
    def _loop():
        topv = topv_ref[...]
        valid = (topv > 0.0).astype(jnp.float32)

        def cond(c):
            _, ch, t = c
            return ch & (t < _K)

        def body(c):
            kv, _, t = c
            sup = jax.lax.dot_general(
                kv, B_ref[...], (((1,), (0,)), ((), ())),
                preferred_element_type=jnp.float32)
            nk = valid * (sup == 0.0).astype(jnp.float32)
            return nk, jnp.any(nk != kv), t + 1

        kv, _, _ = lax.while_loop(
            cond, body, (valid, jnp.bool_(True), jnp.int32(0)))
        keep_ref[...] = kv
        fs_ref[...] = kv * topv


def _nms(tb_col, tc_col, tbT, tcf, topv):
    full = lambda s: pl.BlockSpec(s, lambda i: (0,) * len(s))
    return pl.pallas_call(
        _nms_body,
        grid=(_NBLK + 1,),
        in_specs=[
            full((_KP, 4)),
            full((_KP, 1)),
            full((4, _KP)),
            full((1, _KP)),
            full((1, _KP)),
        ],
        out_specs=[full((1, _KP)), full((1, _KP))],
        out_shape=[
            jax.ShapeDtypeStruct((1, _KP), jnp.float32),
            jax.ShapeDtypeStruct((1, _KP), jnp.float32),
        ],
        scratch_shapes=[pltpu.VMEM((_KP, _KP), jnp.float32)],
    )(tb_col, tc_col, tbT, tcf, topv)


def kernel(prediction, frame_h, frame_w):
    pred = prediction[0]
    packed = _phase_a(pred)
    scores = packed[:, 0]
    topv, topi = lax.top_k(scores, _K)
    g = packed[topi]
    tb = g[:, 2:6]
    tcv = g[:, 1].astype(jnp.int32)
    tb_col = jnp.pad(tb, ((0, _KP - _K), (0, 0)))
    tc_col = jnp.pad(g[:, 1:2], ((0, _KP - _K), (0, 0)))
    topv_row = jnp.pad(topv[None, :], ((0, 0), (0, _KP - _K)),
                       constant_values=-1.0)
    keep, fs = _nms(tb_col, tc_col, tb_col.T, tc_col.T, topv_row)
    keepv = keep[0, :_K]
    det = jnp.concatenate(
        [tb, fs[0, :_K, None], keepv[:, None]], axis=1)
    return det, tcv


# speedup vs baseline: 1.0699x; 1.0699x over previous
import functools

import jax
import jax.numpy as jnp
from jax import lax
from jax.experimental import pallas as pl
from jax.experimental.pallas import tpu as pltpu

_MIN_CONF = 0.25
_IOU_THR = 0.45
_N = 20000
_K = 1000
_KP = 1024
_GAIN = min(640 / 1080, 640 / 1920)
_PAD_X = (640 - 1920 * _GAIN) / 2.0
_PAD_Y = (640 - 1080 * _GAIN) / 2.0


_NTILE = 32
_RBASE = 624
_RSPAN = 656
_NCH = _RSPAN // 16
_MAGIC = 12582912.0


def _sc_phase_a(pred):
    from jax.experimental.pallas import tpu_sc as plsc

    mesh = plsc.VectorSubcoreMesh(core_axis_name="c", subcore_axis_name="s")

    def body(pred_hbm, out_hbm, rows_v, rowout_v):
        wid = lax.axis_index("s") * 2 + lax.axis_index("c")
        base = wid * _RBASE
        pltpu.sync_copy(pred_hbm.at[pl.ds(base * 85, _RSPAN * 85)], rows_v)
        lane = lax.broadcasted_iota(jnp.int32, (16,), 0)

        def chunk(c, carry):
            ids = c * 16 + lane

            flat85 = ids * 85

            def g(col):
                return plsc.load_gather(rows_v, [flat85 + col])

            xc, yc, ww, hh, obj = g(0), g(1), g(2), g(3), g(4)
            m = g(5)
            cid = jnp.zeros((16,), jnp.int32)
            for col in range(6, 85):
                v = g(col)
                gt = v > m
                m = jnp.where(gt, v, m)
                cid = jnp.where(gt, jnp.full((16,), col - 5, jnp.int32), cid)
            conf = obj * m
            cvalid = (cid <= 3) | (cid == 5) | (cid == 7)
            keep = (conf >= _MIN_CONF) & cvalid
            score = jnp.where(keep, conf, -1.0)
            l = (xc - ww * 0.5 - _PAD_X) / _GAIN
            t = (yc - hh * 0.5 - _PAD_Y) / _GAIN
            r = (xc + ww * 0.5 - _PAD_X) / _GAIN
            b = (yc + hh * 0.5 - _PAD_Y) / _GAIN
            flat = ids * 8
            for col, vals in enumerate(
                    [score, cid.astype(jnp.float32),
                     (l + _MAGIC) - _MAGIC, (t + _MAGIC) - _MAGIC,
                     (r + _MAGIC) - _MAGIC, (b + _MAGIC) - _MAGIC]):
                plsc.store_scatter(rowout_v, [flat + col], vals)
            return carry

        lax.fori_loop(0, _NCH, chunk, 0)
        pltpu.sync_copy(rowout_v, out_hbm.at[pl.ds(base * 8, _RSPAN * 8)])

    run = pl.kernel(
        body,
        out_type=jax.ShapeDtypeStruct((_N * 8,), jnp.float32),
        mesh=mesh,
        compiler_params=pltpu.CompilerParams(needs_layout_passes=False),
        scratch_types=[
            pltpu.VMEM((_RSPAN * 85,), jnp.float32),
            pltpu.VMEM((_RSPAN * 8,), jnp.float32),
        ],
    )
    return run(pred.reshape(-1)).reshape(_N, 8)


_NBLK = 8
_BROWS = _KP // _NBLK


def _nms_body(tb_col_ref, tc_col_ref, tbT_ref, tcf_ref, topv_ref,
              keep_ref, fs_ref, B_ref):
    pid = pl.program_id(0)

    @pl.when(pid < _NBLK)
    def _build():
        r0 = pid * _BROWS
        tc_c = tc_col_ref[pl.ds(r0, _BROWS), :] * 10000.0
        Lc = tb_col_ref[pl.ds(r0, _BROWS), 0:1] + tc_c
        Tc = tb_col_ref[pl.ds(r0, _BROWS), 1:2] + tc_c
        Rc = tb_col_ref[pl.ds(r0, _BROWS), 2:3] + tc_c
        Bc = tb_col_ref[pl.ds(r0, _BROWS), 3:4] + tc_c
        area_c = jnp.maximum(Rc - Lc, 0.0) * jnp.maximum(Bc - Tc, 0.0)
        tc_r = tcf_ref[...] * 10000.0
        Lr = tbT_ref[0:1, :] + tc_r
        Tr = tbT_ref[1:2, :] + tc_r
        Rr = tbT_ref[2:3, :] + tc_r
        Br = tbT_ref[3:4, :] + tc_r
        area_r = jnp.maximum(Rr - Lr, 0.0) * jnp.maximum(Br - Tr, 0.0)
        wx = jnp.clip(jnp.minimum(Rc, Rr) - jnp.maximum(Lc, Lr), 0.0, None)
        wy = jnp.clip(jnp.minimum(Bc, Br) - jnp.maximum(Tc, Tr), 0.0, None)
        inter = wx * wy
        iou = inter / (area_c + area_r - inter + 1e-9)
        ii = lax.broadcasted_iota(jnp.int32, (_BROWS, _KP), 0) + r0
        jj = lax.broadcasted_iota(jnp.int32, (_BROWS, _KP), 1)
        B_ref[pl.ds(r0, _BROWS), :] = (
            (iou > _IOU_THR) & (jj > ii)).astype(jnp.float32)

    @pl.when(pid == _NBLK)
    def _loop():
        topv = topv_ref[...]
        valid = (topv > 0.0).astype(jnp.float32)

        def cond(c):
            _, ch, t = c
            return ch & (t < _K)

        def body(c):
            kv, _, t = c
            sup = jax.lax.dot_general(
                kv, B_ref[...], (((1,), (0,)), ((), ())),
                preferred_element_type=jnp.float32)
            nk = valid * (sup == 0.0).astype(jnp.float32)
            return nk, jnp.any(nk != kv), t + 1

        kv, _, _ = lax.while_loop(
            cond, body, (valid, jnp.bool_(True), jnp.int32(0)))
        keep_ref[...] = kv
        fs_ref[...] = kv * topv


def _nms(tb_col, tc_col, tbT, tcf, topv):
    full = lambda s: pl.BlockSpec(s, lambda i: (0,) * len(s))
    return pl.pallas_call(
        _nms_body,
        grid=(_NBLK + 1,),
        in_specs=[
            full((_KP, 4)),
            full((_KP, 1)),
            full((4, _KP)),
            full((1, _KP)),
            full((1, _KP)),
        ],
        out_specs=[full((1, _KP)), full((1, _KP))],
        out_shape=[
            jax.ShapeDtypeStruct((1, _KP), jnp.float32),
            jax.ShapeDtypeStruct((1, _KP), jnp.float32),
        ],
        scratch_shapes=[pltpu.VMEM((_KP, _KP), jnp.float32)],
    )(tb_col, tc_col, tbT, tcf, topv)


def kernel(prediction, frame_h, frame_w):
    pred = prediction[0]
    packed = _sc_phase_a(pred)
    scores = packed[:, 0]
    topv, topi = lax.top_k(scores, _K)
    g = packed[topi]
    tb = g[:, 2:6]
    tcv = g[:, 1].astype(jnp.int32)
    tb_col = jnp.pad(tb, ((0, _KP - _K), (0, 0)))
    tc_col = jnp.pad(g[:, 1:2], ((0, _KP - _K), (0, 0)))
    topv_row = jnp.pad(topv[None, :], ((0, 0), (0, _KP - _K)),
                       constant_values=-1.0)
    keep, fs = _nms(tb_col, tc_col, tb_col.T, tc_col.T, topv_row)
    keepv = keep[0, :_K]
    det = jnp.concatenate(
        [tb, fs[0, :_K, None], keepv[:, None]], axis=1)
    return det, tcv


# speedup vs baseline: 1.0944x; 1.0229x over previous
import functools

import jax
import jax.numpy as jnp
from jax import lax
from jax.experimental import pallas as pl
from jax.experimental.pallas import tpu as pltpu

_MIN_CONF = 0.25
_IOU_THR = 0.45
_N = 20000
_K = 1000
_KP = 1024
_GAIN = min(640 / 1080, 640 / 1920)
_PAD_X = (640 - 1920 * _GAIN) / 2.0
_PAD_Y = (640 - 1080 * _GAIN) / 2.0


_NTILE = 32
_RBASE = 624
_RSPAN = 656
_NCH = _RSPAN // 16
_MAGIC = 12582912.0


def _sc_phase_a(pred):
    from jax.experimental.pallas import tpu_sc as plsc

    mesh = plsc.VectorSubcoreMesh(core_axis_name="c", subcore_axis_name="s")

    def body(pred_hbm, out_hbm, rows_v, rowout_v):
        wid = lax.axis_index("s") * 2 + lax.axis_index("c")
        base = wid * _RBASE
        pltpu.sync_copy(pred_hbm.at[pl.ds(base * 85, _RSPAN * 85)], rows_v)
        lane = lax.broadcasted_iota(jnp.int32, (16,), 0)

        def chunk(c, carry):
            ids = c * 16 + lane

            flat85 = ids * 85

            def g(col):
                return plsc.load_gather(rows_v, [flat85 + col])

            xc, yc, ww, hh, obj = g(0), g(1), g(2), g(3), g(4)
            vals = [g(col) for col in range(5, 85)]
            idxs = [jnp.full((16,), col, jnp.int32) for col in range(80)]
            while len(vals) > 1:
                nv, ni = [], []
                for k in range(0, len(vals) - 1, 2):
                    gt = vals[k + 1] > vals[k]
                    nv.append(jnp.where(gt, vals[k + 1], vals[k]))
                    ni.append(jnp.where(gt, idxs[k + 1], idxs[k]))
                if len(vals) % 2:
                    nv.append(vals[-1])
                    ni.append(idxs[-1])
                vals, idxs = nv, ni
            m, cid = vals[0], idxs[0]
            conf = obj * m
            cvalid = (cid <= 3) | (cid == 5) | (cid == 7)
            keep = (conf >= _MIN_CONF) & cvalid
            score = jnp.where(keep, conf, -1.0)
            l = (xc - ww * 0.5 - _PAD_X) / _GAIN
            t = (yc - hh * 0.5 - _PAD_Y) / _GAIN
            r = (xc + ww * 0.5 - _PAD_X) / _GAIN
            b = (yc + hh * 0.5 - _PAD_Y) / _GAIN
            flat = ids * 8
            for col, vals in enumerate(
                    [score, cid.astype(jnp.float32),
                     (l + _MAGIC) - _MAGIC, (t + _MAGIC) - _MAGIC,
                     (r + _MAGIC) - _MAGIC, (b + _MAGIC) - _MAGIC]):
                plsc.store_scatter(rowout_v, [flat + col], vals)
            return carry

        lax.fori_loop(0, _NCH, chunk, 0)
        pltpu.sync_copy(rowout_v, out_hbm.at[pl.ds(base * 8, _RSPAN * 8)])

    run = pl.kernel(
        body,
        out_type=jax.ShapeDtypeStruct((_N * 8,), jnp.float32),
        mesh=mesh,
        compiler_params=pltpu.CompilerParams(needs_layout_passes=False),
        scratch_types=[
            pltpu.VMEM((_RSPAN * 85,), jnp.float32),
            pltpu.VMEM((_RSPAN * 8,), jnp.float32),
        ],
    )
    return run(pred.reshape(-1)).reshape(_N, 8)


_NBLK = 8
_BROWS = _KP // _NBLK


def _nms_body(tb_col_ref, tc_col_ref, tbT_ref, tcf_ref, topv_ref,
              keep_ref, fs_ref, B_ref):
    pid = pl.program_id(0)

    @pl.when(pid < _NBLK)
    def _build():
        r0 = pid * _BROWS
        tc_c = tc_col_ref[pl.ds(r0, _BROWS), :] * 10000.0
        Lc = tb_col_ref[pl.ds(r0, _BROWS), 0:1] + tc_c
        Tc = tb_col_ref[pl.ds(r0, _BROWS), 1:2] + tc_c
        Rc = tb_col_ref[pl.ds(r0, _BROWS), 2:3] + tc_c
        Bc = tb_col_ref[pl.ds(r0, _BROWS), 3:4] + tc_c
        area_c = jnp.maximum(Rc - Lc, 0.0) * jnp.maximum(Bc - Tc, 0.0)
        tc_r = tcf_ref[...] * 10000.0
        Lr = tbT_ref[0:1, :] + tc_r
        Tr = tbT_ref[1:2, :] + tc_r
        Rr = tbT_ref[2:3, :] + tc_r
        Br = tbT_ref[3:4, :] + tc_r
        area_r = jnp.maximum(Rr - Lr, 0.0) * jnp.maximum(Br - Tr, 0.0)
        wx = jnp.clip(jnp.minimum(Rc, Rr) - jnp.maximum(Lc, Lr), 0.0, None)
        wy = jnp.clip(jnp.minimum(Bc, Br) - jnp.maximum(Tc, Tr), 0.0, None)
        inter = wx * wy
        iou = inter / (area_c + area_r - inter + 1e-9)
        ii = lax.broadcasted_iota(jnp.int32, (_BROWS, _KP), 0) + r0
        jj = lax.broadcasted_iota(jnp.int32, (_BROWS, _KP), 1)
        B_ref[pl.ds(r0, _BROWS), :] = (
            (iou > _IOU_THR) & (jj > ii)).astype(jnp.float32)

    @pl.when(pid == _NBLK)
    def _loop():
        topv = topv_ref[...]
        valid = (topv > 0.0).astype(jnp.float32)

        def cond(c):
            _, ch, t = c
            return ch & (t < _K)

        def body(c):
            kv, _, t = c
            sup = jax.lax.dot_general(
                kv, B_ref[...], (((1,), (0,)), ((), ())),
                preferred_element_type=jnp.float32)
            nk = valid * (sup == 0.0).astype(jnp.float32)
            return nk, jnp.any(nk != kv), t + 1

        kv, _, _ = lax.while_loop(
            cond, body, (valid, jnp.bool_(True), jnp.int32(0)))
        keep_ref[...] = kv
        fs_ref[...] = kv * topv


def _nms(tb_col, tc_col, tbT, tcf, topv):
    full = lambda s: pl.BlockSpec(s, lambda i: (0,) * len(s))
    return pl.pallas_call(
        _nms_body,
        grid=(_NBLK + 1,),
        in_specs=[
            full((_KP, 4)),
            full((_KP, 1)),
            full((4, _KP)),
            full((1, _KP)),
            full((1, _KP)),
        ],
        out_specs=[full((1, _KP)), full((1, _KP))],
        out_shape=[
            jax.ShapeDtypeStruct((1, _KP), jnp.float32),
            jax.ShapeDtypeStruct((1, _KP), jnp.float32),
        ],
        scratch_shapes=[pltpu.VMEM((_KP, _KP), jnp.float32)],
    )(tb_col, tc_col, tbT, tcf, topv)


def kernel(prediction, frame_h, frame_w):
    pred = prediction[0]
    packed = _sc_phase_a(pred)
    scores = packed[:, 0]
    topv, topi = lax.top_k(scores, _K)
    g = packed[topi]
    tb = g[:, 2:6]
    tcv = g[:, 1].astype(jnp.int32)
    tb_col = jnp.pad(tb, ((0, _KP - _K), (0, 0)))
    tc_col = jnp.pad(g[:, 1:2], ((0, _KP - _K), (0, 0)))
    topv_row = jnp.pad(topv[None, :], ((0, 0), (0, _KP - _K)),
                       constant_values=-1.0)
    keep, fs = _nms(tb_col, tc_col, tb_col.T, tc_col.T, topv_row)
    keepv = keep[0, :_K]
    det = jnp.concatenate(
        [tb, fs[0, :_K, None], keepv[:, None]], axis=1)
    return det, tcv
